# baseline (speedup 1.0000x reference)
import functools

import jax
import jax.numpy as jnp
from jax import lax
from jax.experimental import pallas as pl
from jax.experimental.pallas import tpu as pltpu
from jax.experimental.pallas import tpu_sc as plsc


def kernel(tokens, embed):
    B, S = tokens.shape
    V, D = embed.shape
    N = B * S

    info = plsc.get_sparse_core_info()
    NC, NS = info.num_cores, info.num_subcores
    NW = NC * NS
    CH = 128
    per_w = N // NW
    n_ch = per_w // CH

    tok2d = tokens.reshape(N // CH, CH).astype(jnp.int32)

    mesh = plsc.VectorSubcoreMesh(core_axis_name="c", subcore_axis_name="s")

    NBUF = 4
    LAG = 3
    n_grp = n_ch // NBUF

    @functools.partial(
        pl.kernel,
        mesh=mesh,
        out_type=jax.ShapeDtypeStruct((N, D), jnp.float32),
        scratch_types=[
            pltpu.VMEM((n_ch, CH), jnp.int32),
            pltpu.VMEM((NBUF, CH, D), jnp.float32),
        ] + [pltpu.SemaphoreType.DMA] * (2 * NBUF),
    )
    def gather_kernel(tok_hbm, tab_hbm, out_hbm, idx_v, rows_v, *sems):
        gsem, ssem = sems[:NBUF], sems[NBUF:]
        wid = lax.axis_index("s") * NC + lax.axis_index("c")
        row0 = wid * n_ch
        pltpu.sync_copy(tok_hbm.at[pl.ds(row0, n_ch)], idx_v)

        def g_copy(j, b):
            return pltpu.make_async_copy(
                tab_hbm.at[idx_v.at[j]], rows_v.at[b], gsem[b])

        def s_copy(j, b):
            return pltpu.make_async_copy(
                rows_v.at[b], out_hbm.at[pl.ds((row0 + j) * CH, CH)], ssem[b])

        for j in range(LAG):
            g_copy(j, j % NBUF).start()

        def body(g, carry):
            for b in range(NBUF):
                j = g * NBUF + b
                g_copy(j, b).wait()
                s_copy(j, b).start()
                jp = j + LAG
                bp = (b + LAG) % NBUF

                @pl.when(jp < n_ch)
                def _():
                    @pl.when(jp >= NBUF)
                    def _():
                        s_copy(jp - NBUF, bp).wait()
                    g_copy(jp, bp).start()
            return carry

        lax.fori_loop(0, n_grp, body, 0)

        for b in range(NBUF):
            s_copy(n_ch - NBUF + b, b).wait()

    out = gather_kernel(tok2d, embed)
    return out.reshape(B, S, D)


# speedup vs baseline: 2.0143x; 2.0143x over previous
import functools

import jax
import jax.numpy as jnp
from jax import lax
from jax.experimental import pallas as pl
from jax.experimental.pallas import tpu as pltpu
from jax.experimental.pallas import tpu_sc as plsc


def kernel(tokens, embed):
    B, S = tokens.shape
    V, D = embed.shape
    N = B * S

    info = plsc.get_sparse_core_info()
    NC, NS = info.num_cores, info.num_subcores
    NW = NC * NS
    CH = 128
    per_w = N // NW
    n_ch = per_w // CH

    tok2d = tokens.reshape(N // CH, CH).astype(jnp.int32)

    mesh = plsc.VectorSubcoreMesh(core_axis_name="c", subcore_axis_name="s")

    NBUF = 4
    LAG = 3
    n_grp = n_ch // NBUF

    @functools.partial(
        pl.kernel,
        mesh=mesh,
        out_type=jax.ShapeDtypeStruct((N, D), jnp.float32),
        scratch_types=[
            pltpu.VMEM((n_ch, CH), jnp.int32),
            pltpu.VMEM((NBUF, CH, D), jnp.float32),
            pltpu.VMEM_SHARED((V, D), jnp.float32),
        ] + [pltpu.SemaphoreType.DMA] * (2 * NBUF),
    )
    def gather_kernel(tok_hbm, tab_hbm, out_hbm, idx_v, rows_v, tab_sp, *sems):
        gsem, ssem = sems[:NBUF], sems[NBUF:]
        sid = lax.axis_index("s")
        wid = sid * NC + lax.axis_index("c")
        row0 = wid * n_ch

        @pl.when(sid == 0)
        def _():
            pltpu.sync_copy(tab_hbm, tab_sp)

        pltpu.sync_copy(tok_hbm.at[pl.ds(row0, n_ch)], idx_v)
        plsc.subcore_barrier()

        def g_copy(j, b):
            return pltpu.make_async_copy(
                tab_sp.at[idx_v.at[j]], rows_v.at[b], gsem[b])

        def s_copy(j, b):
            return pltpu.make_async_copy(
                rows_v.at[b], out_hbm.at[pl.ds((row0 + j) * CH, CH)], ssem[b])

        for j in range(LAG):
            g_copy(j, j % NBUF).start()

        def body(g, carry):
            for b in range(NBUF):
                j = g * NBUF + b
                g_copy(j, b).wait()
                s_copy(j, b).start()
                jp = j + LAG
                bp = (b + LAG) % NBUF

                @pl.when(jp < n_ch)
                def _():
                    @pl.when(jp >= NBUF)
                    def _():
                        s_copy(jp - NBUF, bp).wait()
                    g_copy(jp, bp).start()
            return carry

        lax.fori_loop(0, n_grp, body, 0)

        for b in range(NBUF):
            s_copy(n_ch - NBUF + b, b).wait()

    out = gather_kernel(tok2d, embed)
    return out.reshape(B, S, D)
